# baseline (speedup 1.0000x reference)
import dataclasses
import functools

import jax
import jax.numpy as jnp
import numpy as np
from jax import lax
from jax.experimental import pallas as pl
from jax.experimental.pallas import tpu as pltpu
from jax.experimental.pallas import tpu_sc as plsc

_NC = 2
_NS = 16
_NW = _NC * _NS
_LANES = 16
_H = 4096
_SB = 4 * _H
_S1 = _H.bit_length() - 1
_HIMASK = np.int32(-65536)


def _bf16_bits(x):
    u = lax.bitcast_convert_type(x, jnp.int32)
    bias = np.int32(0x7FFF) + ((u >> 16) & 1)
    return u + bias


def _repack_table(table):
    V, D = table.shape
    tt = table.T
    grid = pl.cdiv(V, _SB)
    DW = D // 2

    d = np.arange(D)[:, None]
    k = np.arange(2 * DW)[None, :]
    sel_np = np.where(k < DW, d == 2 * k, d == 2 * (k - DW) + 1)
    sel_f = jnp.asarray(sel_np.astype(np.float32), jnp.bfloat16)

    def body(x_ref, sel_ref, o_ref):
        x = x_ref[...].astype(jnp.bfloat16)
        sel = sel_ref[...]
        parts = []
        for c in range(_SB // _H):
            xc = x[:, c * _H:(c + 1) * _H]
            xeo = lax.dot_general(xc, sel, (((0,), (0,)), ((), ())),
                                  preferred_element_type=jnp.float32)
            u = lax.bitcast_convert_type(xeo, jnp.int32)
            lo = (u[:, :DW] >> 16) & np.int32(0xFFFF)
            hi = u[:, DW:] & _HIMASK
            parts.append(lo | hi)
        o_ref[...] = lax.bitcast_convert_type(
            jnp.concatenate(parts, axis=1), jnp.float32)

    return pl.pallas_call(
        body,
        grid=(grid,),
        in_specs=[pl.BlockSpec((D, _SB), lambda i: (0, i)),
                  pl.BlockSpec((D, 2 * DW), lambda i: (0, 0))],
        out_specs=pl.BlockSpec((_H, 128), lambda i: (i, 0)),
        out_shape=jax.ShapeDtypeStruct((grid * _H, 128), jnp.float32),
    )(tt, sel_f)


def _remap_indices(v):
    return (((v >> (_S1 + 2)) << (_S1 + 2)) + ((v & (_H - 1)) << 2)
            + ((v >> _S1) & 3))


def _make_pool(B, S, DW):
    GS = (128, S - 128)
    RPW = B // _NW
    CH = 64
    NCH = RPW // CH
    NGR = CH * 2
    NBUF = 4
    UNR = 8
    NK = DW // _LANES
    D = 2 * DW
    inv_s = 1.0 / S

    mesh = plsc.VectorSubcoreMesh(core_axis_name="c", subcore_axis_name="s")
    cp = pltpu.CompilerParams(use_tc_tiling_on_sc=False)
    if "needs_layout_passes" in pltpu.CompilerParams.__dataclass_fields__:
        cp = dataclasses.replace(cp, needs_layout_passes=False)

    @functools.partial(
        pl.kernel,
        out_type=jax.ShapeDtypeStruct((B, 2 * D), jnp.float32),
        mesh=mesh,
        compiler_params=cp,
        scratch_types=[
            pltpu.VMEM((2, CH, S), jnp.int32),
            *[pltpu.VMEM((GS[b % 2], DW), jnp.float32) for b in range(NBUF)],
            pltpu.VMEM((RPW, 2 * D), jnp.float32),
            *[pltpu.SemaphoreType.DMA for _ in range(NBUF)],
            pltpu.SemaphoreType.DMA,
        ],
    )
    def pool(text_hbm, table_hbm, out_hbm, idx_v, g0, g1, g2, g3, out_v,
             s0, s1, s2, s3, si):
        bufs = (g0, g1, g2, g3)
        sems = (s0, s1, s2, s3)
        offs = (0, GS[0])
        wid = lax.axis_index("s") * _NC + lax.axis_index("c")
        base = wid * RPW

        def idx_copy(ci, slot):
            return pltpu.make_async_copy(
                text_hbm.at[pl.ds(base + ci * CH, CH)], idx_v.at[slot], si)

        def gather_start(g, b, slot):
            half = b % 2
            row = g // 2
            src = table_hbm.at[
                idx_v.at[slot, row, pl.ds(offs[half], GS[half])]]
            pltpu.make_async_copy(src, bufs[b], sems[b]).start()

        def gather_wait(b):
            half = b % 2
            src = table_hbm.at[idx_v.at[0, 0, pl.ds(offs[half], GS[half])]]
            pltpu.make_async_copy(src, bufs[b], sems[b]).wait()

        def accum(buf, n):
            def body(k, carry):
                out = list(carry)
                for r in range(UNR):
                    row = k * UNR + r
                    p = 4 * (r % 2)
                    for k2 in range(NK):
                        w = plsc.bitcast(
                            buf[row, pl.ds(_LANES * k2, _LANES)], jnp.int32)
                        lo = plsc.bitcast(w << 16, jnp.float32)
                        hi = plsc.bitcast(w & _HIMASK, jnp.float32)
                        out[p + 2 * k2] = out[p + 2 * k2] + lo
                        out[p + 2 * k2 + 1] = out[p + 2 * k2 + 1] + hi
                return tuple(out)

            zeros = tuple(jnp.zeros((_LANES,), jnp.float32)
                          for _ in range(4 * NK))
            accs = lax.fori_loop(0, n // UNR, body, zeros)
            return [accs[j] + accs[4 + j] for j in range(2 * NK)]

        def consume(g, b, ci):
            gather_wait(b)
            accs = accum(bufs[b], GS[b % 2])
            out_r = ci * CH + g // 2
            if b % 2 == 0:
                for j in range(2 * NK):
                    out_v[out_r, pl.ds(j * _LANES, _LANES)] = accs[j] * inv_s
            else:
                for j in range(2 * NK):
                    sl = pl.ds(j * _LANES, _LANES)
                    out_v[out_r, sl] = out_v[out_r, sl] + accs[j] * inv_s

        @pl.loop(0, RPW)
        def _zero(r):
            for j in range(2 * NK, 4 * NK):
                out_v[r, pl.ds(j * _LANES, _LANES)] = jnp.zeros(
                    (_LANES,), jnp.float32)

        idx_copy(0, 0).start()

        @pl.loop(0, NCH)
        def _chunk(ci):
            slot = lax.rem(ci, 2)
            idx_copy(0, 0).wait()
            for b in range(NBUF):
                gather_start(b, b, slot)

            @pl.when(ci + 1 < NCH)
            def _():
                idx_copy(ci + 1, 1 - slot).start()

            @pl.loop(0, (NGR - NBUF) // NBUF)
            def _grp(i):
                for b in range(NBUF):
                    g = i * NBUF + b
                    consume(g, b, ci)
                    gather_start(g + NBUF, b, slot)

            for b in range(NBUF):
                consume(NGR - NBUF + b, b, ci)

        pltpu.sync_copy(out_v, out_hbm.at[pl.ds(base, RPW)])

    return pool


def _pool_perm(D):
    perm = []
    for k in range(D // 32):
        perm.extend(range(32 * k, 32 * k + 32, 2))
        perm.extend(range(32 * k + 1, 32 * k + 32, 2))
    return perm


def _linear_t(pooled, Wt, bc):
    B, D2 = pooled.shape
    D = D2 // 2
    N = Wt.shape[1]
    BLK = 4096

    def body(w_ref, x_ref, b_ref, o_ref):
        x = x_ref[...][:, :D]
        o_ref[...] = lax.dot_general(
            w_ref[...], x, (((0,), (1,)), ((), ())),
            preferred_element_type=jnp.float32,
        ) + b_ref[...]

    return pl.pallas_call(
        body,
        grid=(B // BLK,),
        in_specs=[
            pl.BlockSpec((D, N), lambda i: (0, 0)),
            pl.BlockSpec((BLK, D2), lambda i: (i, 0)),
            pl.BlockSpec((N, 1), lambda i: (0, 0)),
        ],
        out_specs=pl.BlockSpec((N, BLK), lambda i: (0, i)),
        out_shape=jax.ShapeDtypeStruct((N, B), jnp.float32),
    )(Wt, pooled, bc)


def kernel(text, table, W, b):
    B, S = text.shape
    V, D = table.shape
    text2 = _remap_indices(text.astype(jnp.int32))
    packed = _repack_table(table)
    Vp = packed.shape[0] * packed.shape[1] // (D // 2)
    sc_table = packed.reshape(-1).reshape(Vp, D // 2)
    pooled = _make_pool(B, S, D // 2)(text2, sc_table)
    Wtp = jnp.take(W.T, np.asarray(_pool_perm(D), np.int32), axis=0)
    logits_t = _linear_t(pooled, Wtp, b.reshape(-1, 1))
    return logits_t.T


# speedup vs baseline: 1.0109x; 1.0109x over previous
import dataclasses
import functools

import jax
import jax.numpy as jnp
import numpy as np
from jax import lax
from jax.experimental import pallas as pl
from jax.experimental.pallas import tpu as pltpu
from jax.experimental.pallas import tpu_sc as plsc

_NC = 2
_NS = 16
_NW = _NC * _NS
_LANES = 16
_H = 8192
_SB = 4 * _H
_S1 = _H.bit_length() - 1
_HIMASK = np.int32(-65536)


def _bf16_bits(x):
    u = lax.bitcast_convert_type(x, jnp.int32)
    bias = np.int32(0x7FFF) + ((u >> 16) & 1)
    return u + bias


def _repack_table(table):
    V, D = table.shape
    tt = table.T
    grid = pl.cdiv(V, _SB)
    DW = D // 2

    d = np.arange(D)[:, None]
    k = np.arange(2 * DW)[None, :]
    sel_np = np.where(k < DW, d == 2 * k, d == 2 * (k - DW) + 1)
    sel_f = jnp.asarray(sel_np.astype(np.float32), jnp.bfloat16)

    def body(x_ref, sel_ref, o_ref):
        x = x_ref[...].astype(jnp.bfloat16)
        sel = sel_ref[...]
        parts = []
        for c in range(_SB // _H):
            xc = x[:, c * _H:(c + 1) * _H]
            xeo = lax.dot_general(xc, sel, (((0,), (0,)), ((), ())),
                                  preferred_element_type=jnp.float32)
            u = lax.bitcast_convert_type(xeo, jnp.int32)
            lo = (u[:, :DW] >> 16) & np.int32(0xFFFF)
            hi = u[:, DW:] & _HIMASK
            parts.append(lo | hi)
        o_ref[...] = lax.bitcast_convert_type(
            jnp.concatenate(parts, axis=1), jnp.float32)

    return pl.pallas_call(
        body,
        grid=(grid,),
        in_specs=[pl.BlockSpec((D, _SB), lambda i: (0, i)),
                  pl.BlockSpec((D, 2 * DW), lambda i: (0, 0))],
        out_specs=pl.BlockSpec((_H, 128), lambda i: (i, 0)),
        out_shape=jax.ShapeDtypeStruct((grid * _H, 128), jnp.float32),
    )(tt, sel_f)


def _remap_indices(v):
    return (((v >> (_S1 + 2)) << (_S1 + 2)) + ((v & (_H - 1)) << 2)
            + ((v >> _S1) & 3))


def _make_pool(B, S, DW):
    GS = (128, S - 128)
    RPW = B // _NW
    CH = 64
    NCH = RPW // CH
    NGR = CH * 2
    NBUF = 4
    UNR = 8
    NK = DW // _LANES
    D = 2 * DW
    inv_s = 1.0 / S

    mesh = plsc.VectorSubcoreMesh(core_axis_name="c", subcore_axis_name="s")
    cp = pltpu.CompilerParams(use_tc_tiling_on_sc=False)
    if "needs_layout_passes" in pltpu.CompilerParams.__dataclass_fields__:
        cp = dataclasses.replace(cp, needs_layout_passes=False)

    @functools.partial(
        pl.kernel,
        out_type=jax.ShapeDtypeStruct((B, 2 * D), jnp.float32),
        mesh=mesh,
        compiler_params=cp,
        scratch_types=[
            pltpu.VMEM((2, CH, S), jnp.int32),
            *[pltpu.VMEM((GS[b % 2], DW), jnp.float32) for b in range(NBUF)],
            pltpu.VMEM((RPW, 2 * D), jnp.float32),
            *[pltpu.SemaphoreType.DMA for _ in range(NBUF)],
            pltpu.SemaphoreType.DMA,
        ],
    )
    def pool(text_hbm, table_hbm, out_hbm, idx_v, g0, g1, g2, g3, out_v,
             s0, s1, s2, s3, si):
        bufs = (g0, g1, g2, g3)
        sems = (s0, s1, s2, s3)
        offs = (0, GS[0])
        wid = lax.axis_index("s") * _NC + lax.axis_index("c")
        base = wid * RPW

        def idx_copy(ci, slot):
            return pltpu.make_async_copy(
                text_hbm.at[pl.ds(base + ci * CH, CH)], idx_v.at[slot], si)

        def gather_start(g, b, slot):
            half = b % 2
            row = g // 2
            src = table_hbm.at[
                idx_v.at[slot, row, pl.ds(offs[half], GS[half])]]
            pltpu.make_async_copy(src, bufs[b], sems[b]).start()

        def gather_wait(b):
            half = b % 2
            src = table_hbm.at[idx_v.at[0, 0, pl.ds(offs[half], GS[half])]]
            pltpu.make_async_copy(src, bufs[b], sems[b]).wait()

        def accum(buf, n):
            def body(k, carry):
                out = list(carry)
                for r in range(UNR):
                    row = k * UNR + r
                    p = 4 * (r % 2)
                    for k2 in range(NK):
                        w = plsc.bitcast(
                            buf[row, pl.ds(_LANES * k2, _LANES)], jnp.int32)
                        lo = plsc.bitcast(w << 16, jnp.float32)
                        hi = plsc.bitcast(w & _HIMASK, jnp.float32)
                        out[p + 2 * k2] = out[p + 2 * k2] + lo
                        out[p + 2 * k2 + 1] = out[p + 2 * k2 + 1] + hi
                return tuple(out)

            zeros = tuple(jnp.zeros((_LANES,), jnp.float32)
                          for _ in range(4 * NK))
            accs = lax.fori_loop(0, n // UNR, body, zeros)
            return [accs[j] + accs[4 + j] for j in range(2 * NK)]

        def consume(g, b, ci):
            gather_wait(b)
            accs = accum(bufs[b], GS[b % 2])
            out_r = ci * CH + g // 2
            if b % 2 == 0:
                for j in range(2 * NK):
                    out_v[out_r, pl.ds(j * _LANES, _LANES)] = accs[j] * inv_s
            else:
                for j in range(2 * NK):
                    sl = pl.ds(j * _LANES, _LANES)
                    out_v[out_r, sl] = out_v[out_r, sl] + accs[j] * inv_s

        @pl.loop(0, RPW)
        def _zero(r):
            for j in range(2 * NK, 4 * NK):
                out_v[r, pl.ds(j * _LANES, _LANES)] = jnp.zeros(
                    (_LANES,), jnp.float32)

        idx_copy(0, 0).start()

        @pl.loop(0, NCH)
        def _chunk(ci):
            slot = lax.rem(ci, 2)
            idx_copy(0, 0).wait()
            for b in range(NBUF):
                gather_start(b, b, slot)

            @pl.when(ci + 1 < NCH)
            def _():
                idx_copy(ci + 1, 1 - slot).start()

            @pl.loop(0, (NGR - NBUF) // NBUF)
            def _grp(i):
                for b in range(NBUF):
                    g = i * NBUF + b
                    consume(g, b, ci)
                    gather_start(g + NBUF, b, slot)

            for b in range(NBUF):
                consume(NGR - NBUF + b, b, ci)

        pltpu.sync_copy(out_v, out_hbm.at[pl.ds(base, RPW)])

    return pool


def _pool_perm(D):
    perm = []
    for k in range(D // 32):
        perm.extend(range(32 * k, 32 * k + 32, 2))
        perm.extend(range(32 * k + 1, 32 * k + 32, 2))
    return perm


def _linear_t(pooled, Wt, bc):
    B, D2 = pooled.shape
    D = D2 // 2
    N = Wt.shape[1]
    BLK = 4096

    def body(w_ref, x_ref, b_ref, o_ref):
        x = x_ref[...][:, :D]
        o_ref[...] = lax.dot_general(
            w_ref[...], x, (((0,), (1,)), ((), ())),
            preferred_element_type=jnp.float32,
        ) + b_ref[...]

    return pl.pallas_call(
        body,
        grid=(B // BLK,),
        in_specs=[
            pl.BlockSpec((D, N), lambda i: (0, 0)),
            pl.BlockSpec((BLK, D2), lambda i: (i, 0)),
            pl.BlockSpec((N, 1), lambda i: (0, 0)),
        ],
        out_specs=pl.BlockSpec((N, BLK), lambda i: (0, i)),
        out_shape=jax.ShapeDtypeStruct((N, B), jnp.float32),
    )(Wt, pooled, bc)


def kernel(text, table, W, b):
    B, S = text.shape
    V, D = table.shape
    text2 = _remap_indices(text.astype(jnp.int32))
    packed = _repack_table(table)
    Vp = packed.shape[0] * packed.shape[1] // (D // 2)
    sc_table = packed.reshape(-1).reshape(Vp, D // 2)
    pooled = _make_pool(B, S, D // 2)(text2, sc_table)
    Wtp = jnp.take(W.T, np.asarray(_pool_perm(D), np.int32), axis=0)
    logits_t = _linear_t(pooled, Wtp, b.reshape(-1, 1))
    return logits_t.T
